# baseline (speedup 1.0000x reference)
import functools

import jax
import jax.numpy as jnp
from jax import lax
from jax.experimental import pallas as pl
from jax.experimental.pallas import tpu as pltpu
from jax.experimental.pallas import tpu_sc as plsc

E = 8
T = 4096
D = 768
TILE = 1024
NT = T // TILE
S = NT + E - 1


def _sc_permute_rows(table, idx, direction):
    info = plsc.get_sparse_core_info()
    nw = info.num_cores * info.num_subcores
    bpw = T // nw
    mesh = plsc.VectorSubcoreMesh(core_axis_name="c", subcore_axis_name="s")

    @functools.partial(
        pl.kernel,
        mesh=mesh,
        out_type=jax.ShapeDtypeStruct((T, D), jnp.float32),
        scratch_types=[
            pltpu.VMEM((bpw,), jnp.int32),
            pltpu.VMEM((bpw, D), jnp.float32),
            pltpu.SemaphoreType.DMA,
        ],
    )
    def permute_kernel(table_hbm, idx_hbm, out_hbm, idx_v, rows_v, sem):
        wid = lax.axis_index("s") * info.num_cores + lax.axis_index("c")
        base = wid * bpw
        pltpu.sync_copy(idx_hbm.at[pl.ds(base, bpw)], idx_v)
        if direction == "gather":
            pltpu.async_copy(table_hbm.at[idx_v], rows_v, sem).wait()
            pltpu.sync_copy(rows_v, out_hbm.at[pl.ds(base, bpw)])
        else:
            pltpu.sync_copy(table_hbm.at[pl.ds(base, bpw)], rows_v)
            pltpu.async_copy(rows_v, out_hbm.at[idx_v], sem).wait()

    return permute_kernel(table, idx)


def _mm_body(meta_ref, x_ref, w_ref, b_ref, out_ref):
    s = pl.program_id(0)
    t = meta_ref[0, s]
    lo = meta_ref[2, s]
    hi = meta_ref[3, s]
    fresh = meta_ref[4, s] == 1
    full = (lo == t * TILE) & (hi == (t + 1) * TILE)
    acc = jnp.dot(
        x_ref[...].astype(jnp.bfloat16),
        w_ref[0].astype(jnp.bfloat16),
        preferred_element_type=jnp.float32,
    ) + b_ref[0]

    def masked():
        grow = t * TILE + lax.broadcasted_iota(jnp.int32, (TILE, 1), 0)
        return jnp.where((grow >= lo) & (grow < hi), acc, 0.0)

    @pl.when(fresh & full)
    def _init_full():
        out_ref[...] = acc

    @pl.when(fresh & ~full)
    def _init_partial():
        out_ref[...] = masked()

    @pl.when(~fresh)
    def _accum():
        out_ref[...] = out_ref[...] + masked()


def _grouped_matmul(x_sorted, W, b, meta):
    grid_spec = pltpu.PrefetchScalarGridSpec(
        num_scalar_prefetch=1,
        grid=(S,),
        in_specs=[
            pl.BlockSpec((TILE, D), lambda s, m: (m[0, s], 0)),
            pl.BlockSpec((1, D, D), lambda s, m: (m[1, s], 0, 0)),
            pl.BlockSpec((1, 1, D), lambda s, m: (m[1, s], 0, 0)),
        ],
        out_specs=pl.BlockSpec((TILE, D), lambda s, m: (m[0, s], 0)),
    )
    return pl.pallas_call(
        _mm_body,
        grid_spec=grid_spec,
        out_shape=jax.ShapeDtypeStruct((T, D), jnp.float32),
    )(meta, x_sorted, W, b.reshape(E, 1, D))


def _meta_body(g_ref, inv_ref, meta_ref):
    gt = g_ref[...]
    rows = lax.broadcasted_iota(jnp.int32, (128, 128), 0)
    cols = lax.broadcasted_iota(jnp.int32, (128, 128), 1)
    tri = (rows >= cols).astype(jnp.float32)
    rc = lax.broadcasted_iota(jnp.int32, (32, 32), 0)
    cc = lax.broadcasted_iota(jnp.int32, (32, 32), 1)
    tric = (rc < cc).astype(jnp.float32)

    inv_acc = jnp.zeros((128, 32), jnp.float32)
    start = []
    count = []
    run = jnp.int32(0)
    for e in range(E):
        oh = (gt == e).astype(jnp.float32)
        cs = jnp.dot(tri, oh, preferred_element_type=jnp.float32)
        cnt = cs[127:128, :]
        pre = jnp.dot(cnt, tric, preferred_element_type=jnp.float32)
        tot = jnp.sum(cnt).astype(jnp.int32)
        start.append(run)
        count.append(tot)
        run = run + tot
        inv_acc = inv_acc + oh * (jnp.float32(1.0) * start[e] + pre + cs - 1.0)
    inv_ref[...] = inv_acc.astype(jnp.int32)

    def schedule(p):
        eid = jnp.zeros((1, 16), jnp.int32)
        cum = jnp.int32(0)
        nvs = []
        for e in range(E):
            s_e, c_e = start[e], count[e]
            ft = lax.div(s_e, jnp.int32(TILE))
            lt = lax.div(s_e + c_e - 1, jnp.int32(TILE))
            nv = jnp.where(c_e > 0, lt - ft + 1, 0)
            nvs.append((ft, nv))
            cum = cum + nv
            if e < E - 1:
                eid = eid + jnp.where(cum <= p, 1, 0)
        tile = jnp.zeros((1, 16), jnp.int32)
        lo = jnp.zeros((1, 16), jnp.int32)
        hi = jnp.zeros((1, 16), jnp.int32)
        vcum = jnp.int32(0)
        for e in range(E):
            ft, nv = nvs[e]
            m = eid == e
            t_e = ft + (p - vcum)
            tile = jnp.where(m, t_e, tile)
            lo = jnp.where(m, jnp.maximum(start[e], t_e * TILE), lo)
            hi = jnp.where(m, jnp.minimum(start[e] + count[e], (t_e + 1) * TILE), hi)
            vcum = vcum + nv
        valid = p < cum
        tile = jnp.where(valid, tile, NT - 1)
        lo = jnp.where(valid, lo, 0)
        hi = jnp.where(valid, hi, 0)
        return eid, tile, lo, hi

    p = lax.broadcasted_iota(jnp.int32, (1, 16), 1)
    eid, tile, lo, hi = schedule(p)
    _, tile_prev, _, _ = schedule(p - 1)
    fresh = ((p == 0) | (tile != tile_prev)).astype(jnp.int32)
    meta_ref[...] = jnp.concatenate([tile, eid, lo, hi, fresh], axis=0)


def _tc_route_metadata(g):
    inv2d, meta = pl.pallas_call(
        _meta_body,
        in_specs=[pl.BlockSpec((128, 32), lambda: (0, 0))],
        out_specs=[
            pl.BlockSpec((128, 32), lambda: (0, 0)),
            pl.BlockSpec((5, 16), lambda: (0, 0)),
        ],
        out_shape=[
            jax.ShapeDtypeStruct((128, 32), jnp.int32),
            jax.ShapeDtypeStruct((5, 16), jnp.int32),
        ],
    )(g.reshape(128, 32))
    return inv2d.reshape(T), meta


def kernel(x, groups, W, b):
    g = groups[:, 0]
    inv_idx, meta = _tc_route_metadata(g)
    x_sorted = _sc_permute_rows(x, inv_idx, "scatter")
    out_sorted = _grouped_matmul(x_sorted, W, b, meta)
    return _sc_permute_rows(out_sorted, inv_idx, "gather")


# speedup vs baseline: 1.0041x; 1.0041x over previous
import functools

import jax
import jax.numpy as jnp
from jax import lax
from jax.experimental import pallas as pl
from jax.experimental.pallas import tpu as pltpu
from jax.experimental.pallas import tpu_sc as plsc

E = 8
T = 4096
D = 768
TILE = 1024
NT = T // TILE
S = NT + E - 1


def _sc_permute_rows(table, idx, direction):
    info = plsc.get_sparse_core_info()
    nw = info.num_cores * info.num_subcores
    bpw = T // nw
    mesh = plsc.VectorSubcoreMesh(core_axis_name="c", subcore_axis_name="s")

    @functools.partial(
        pl.kernel,
        mesh=mesh,
        out_type=jax.ShapeDtypeStruct((T, D), jnp.float32),
        scratch_types=[
            pltpu.VMEM((bpw,), jnp.int32),
            pltpu.VMEM((bpw, D), jnp.float32),
            pltpu.SemaphoreType.DMA,
        ],
    )
    def permute_kernel(table_hbm, idx_hbm, out_hbm, idx_v, rows_v, sem):
        wid = lax.axis_index("s") * info.num_cores + lax.axis_index("c")
        base = wid * bpw
        pltpu.sync_copy(idx_hbm.at[pl.ds(base, bpw)], idx_v)
        if direction == "gather":
            pltpu.async_copy(table_hbm.at[idx_v], rows_v, sem).wait()
            pltpu.sync_copy(rows_v, out_hbm.at[pl.ds(base, bpw)])
        else:
            pltpu.sync_copy(table_hbm.at[pl.ds(base, bpw)], rows_v)
            pltpu.async_copy(rows_v, out_hbm.at[idx_v], sem).wait()

    return permute_kernel(table, idx)


def _mm_body(meta_ref, x_ref, w_ref, b_ref, out_ref):
    s = pl.program_id(0)
    t = meta_ref[0, s]
    lo = meta_ref[2, s]
    hi = meta_ref[3, s]
    acc = jnp.dot(
        x_ref[...].astype(jnp.bfloat16),
        w_ref[0].astype(jnp.bfloat16),
        preferred_element_type=jnp.float32,
    )
    grow = t * TILE + lax.broadcasted_iota(jnp.int32, (TILE, 1), 0)
    mask = (grow >= lo) & (grow < hi)
    contrib = jnp.where(mask, acc + b_ref[0], 0.0)

    @pl.when(meta_ref[4, s] == 1)
    def _init():
        out_ref[...] = contrib

    @pl.when(meta_ref[4, s] == 0)
    def _accum():
        out_ref[...] = out_ref[...] + contrib


def _grouped_matmul(x_sorted, W, b, meta):
    grid_spec = pltpu.PrefetchScalarGridSpec(
        num_scalar_prefetch=1,
        grid=(S,),
        in_specs=[
            pl.BlockSpec((TILE, D), lambda s, m: (m[0, s], 0)),
            pl.BlockSpec((1, D, D), lambda s, m: (m[1, s], 0, 0)),
            pl.BlockSpec((1, 1, D), lambda s, m: (m[1, s], 0, 0)),
        ],
        out_specs=pl.BlockSpec((TILE, D), lambda s, m: (m[0, s], 0)),
    )
    return pl.pallas_call(
        _mm_body,
        grid_spec=grid_spec,
        out_shape=jax.ShapeDtypeStruct((T, D), jnp.float32),
    )(meta, x_sorted, W, b.reshape(E, 1, D))


def _meta_body(g_ref, inv_ref, meta_ref):
    gt = g_ref[...]
    rows = lax.broadcasted_iota(jnp.int32, (128, 128), 0)
    cols = lax.broadcasted_iota(jnp.int32, (128, 128), 1)
    tri = (rows >= cols).astype(jnp.float32)
    rc = lax.broadcasted_iota(jnp.int32, (32, 32), 0)
    cc = lax.broadcasted_iota(jnp.int32, (32, 32), 1)
    tric = (rc < cc).astype(jnp.float32)

    inv_acc = jnp.zeros((128, 32), jnp.float32)
    start = []
    count = []
    run = jnp.int32(0)
    for e in range(E):
        oh = (gt == e).astype(jnp.float32)
        cs = jnp.dot(tri, oh, preferred_element_type=jnp.float32)
        cnt = cs[127:128, :]
        pre = jnp.dot(cnt, tric, preferred_element_type=jnp.float32)
        tot = jnp.sum(cnt).astype(jnp.int32)
        start.append(run)
        count.append(tot)
        run = run + tot
        inv_acc = inv_acc + oh * (jnp.float32(1.0) * start[e] + pre + cs - 1.0)
    inv_ref[...] = inv_acc.astype(jnp.int32)

    def schedule(p):
        eid = jnp.zeros((1, 16), jnp.int32)
        cum = jnp.int32(0)
        nvs = []
        for e in range(E):
            s_e, c_e = start[e], count[e]
            ft = lax.div(s_e, jnp.int32(TILE))
            lt = lax.div(s_e + c_e - 1, jnp.int32(TILE))
            nv = jnp.where(c_e > 0, lt - ft + 1, 0)
            nvs.append((ft, nv))
            cum = cum + nv
            if e < E - 1:
                eid = eid + jnp.where(cum <= p, 1, 0)
        tile = jnp.zeros((1, 16), jnp.int32)
        lo = jnp.zeros((1, 16), jnp.int32)
        hi = jnp.zeros((1, 16), jnp.int32)
        vcum = jnp.int32(0)
        for e in range(E):
            ft, nv = nvs[e]
            m = eid == e
            t_e = ft + (p - vcum)
            tile = jnp.where(m, t_e, tile)
            lo = jnp.where(m, jnp.maximum(start[e], t_e * TILE), lo)
            hi = jnp.where(m, jnp.minimum(start[e] + count[e], (t_e + 1) * TILE), hi)
            vcum = vcum + nv
        valid = p < cum
        tile = jnp.where(valid, tile, NT - 1)
        lo = jnp.where(valid, lo, 0)
        hi = jnp.where(valid, hi, 0)
        return eid, tile, lo, hi

    p = lax.broadcasted_iota(jnp.int32, (1, 16), 1)
    eid, tile, lo, hi = schedule(p)
    _, tile_prev, _, _ = schedule(p - 1)
    fresh = ((p == 0) | (tile != tile_prev)).astype(jnp.int32)
    meta_ref[...] = jnp.concatenate([tile, eid, lo, hi, fresh], axis=0)


def _tc_route_metadata(g):
    inv2d, meta = pl.pallas_call(
        _meta_body,
        in_specs=[pl.BlockSpec((128, 32), lambda: (0, 0))],
        out_specs=[
            pl.BlockSpec((128, 32), lambda: (0, 0)),
            pl.BlockSpec((5, 16), lambda: (0, 0)),
        ],
        out_shape=[
            jax.ShapeDtypeStruct((128, 32), jnp.int32),
            jax.ShapeDtypeStruct((5, 16), jnp.int32),
        ],
    )(g.reshape(128, 32))
    return inv2d.reshape(T), meta


def kernel(x, groups, W, b):
    g = groups[:, 0]
    inv_idx, meta = _tc_route_metadata(g)
    x_sorted = _sc_permute_rows(x, inv_idx, "scatter")
    out_sorted = _grouped_matmul(x_sorted, W, b, meta)
    return _sc_permute_rows(out_sorted, inv_idx, "gather")


# speedup vs baseline: 1.0261x; 1.0219x over previous
import functools

import jax
import jax.numpy as jnp
from jax import lax
from jax.experimental import pallas as pl
from jax.experimental.pallas import tpu as pltpu
from jax.experimental.pallas import tpu_sc as plsc

E = 8
T = 4096
D = 768
TILE = 512
NT = T // TILE
S = NT + E - 1


def _sc_permute_rows(table, idx, direction):
    info = plsc.get_sparse_core_info()
    nw = info.num_cores * info.num_subcores
    bpw = T // nw
    mesh = plsc.VectorSubcoreMesh(core_axis_name="c", subcore_axis_name="s")

    @functools.partial(
        pl.kernel,
        mesh=mesh,
        out_type=jax.ShapeDtypeStruct((T, D), jnp.float32),
        scratch_types=[
            pltpu.VMEM((bpw,), jnp.int32),
            pltpu.VMEM((bpw, D), jnp.float32),
            pltpu.SemaphoreType.DMA,
        ],
    )
    def permute_kernel(table_hbm, idx_hbm, out_hbm, idx_v, rows_v, sem):
        wid = lax.axis_index("s") * info.num_cores + lax.axis_index("c")
        base = wid * bpw
        pltpu.sync_copy(idx_hbm.at[pl.ds(base, bpw)], idx_v)
        if direction == "gather":
            pltpu.async_copy(table_hbm.at[idx_v], rows_v, sem).wait()
            pltpu.sync_copy(rows_v, out_hbm.at[pl.ds(base, bpw)])
        else:
            pltpu.sync_copy(table_hbm.at[pl.ds(base, bpw)], rows_v)
            pltpu.async_copy(rows_v, out_hbm.at[idx_v], sem).wait()

    return permute_kernel(table, idx)


def _mm_body(meta_ref, x_ref, w_ref, b_ref, out_ref):
    s = pl.program_id(0)
    t = meta_ref[0, s]
    lo = meta_ref[2, s]
    hi = meta_ref[3, s]
    acc = jnp.dot(
        x_ref[...].astype(jnp.bfloat16),
        w_ref[0].astype(jnp.bfloat16),
        preferred_element_type=jnp.float32,
    )
    grow = t * TILE + lax.broadcasted_iota(jnp.int32, (TILE, 1), 0)
    mask = (grow >= lo) & (grow < hi)
    contrib = jnp.where(mask, acc + b_ref[0], 0.0)

    @pl.when(meta_ref[4, s] == 1)
    def _init():
        out_ref[...] = contrib

    @pl.when(meta_ref[4, s] == 0)
    def _accum():
        out_ref[...] = out_ref[...] + contrib


def _grouped_matmul(x_sorted, W, b, meta):
    grid_spec = pltpu.PrefetchScalarGridSpec(
        num_scalar_prefetch=1,
        grid=(S,),
        in_specs=[
            pl.BlockSpec((TILE, D), lambda s, m: (m[0, s], 0)),
            pl.BlockSpec((1, D, D), lambda s, m: (m[1, s], 0, 0)),
            pl.BlockSpec((1, 1, D), lambda s, m: (m[1, s], 0, 0)),
        ],
        out_specs=pl.BlockSpec((TILE, D), lambda s, m: (m[0, s], 0)),
    )
    return pl.pallas_call(
        _mm_body,
        grid_spec=grid_spec,
        out_shape=jax.ShapeDtypeStruct((T, D), jnp.float32),
    )(meta, x_sorted, W, b.reshape(E, 1, D))


def _meta_body(g_ref, inv_ref, meta_ref):
    gt = g_ref[...]
    rows = lax.broadcasted_iota(jnp.int32, (128, 128), 0)
    cols = lax.broadcasted_iota(jnp.int32, (128, 128), 1)
    tri = (rows >= cols).astype(jnp.float32)
    rc = lax.broadcasted_iota(jnp.int32, (32, 32), 0)
    cc = lax.broadcasted_iota(jnp.int32, (32, 32), 1)
    tric = (rc < cc).astype(jnp.float32)

    inv_acc = jnp.zeros((128, 32), jnp.float32)
    start = []
    count = []
    run = jnp.int32(0)
    for e in range(E):
        oh = (gt == e).astype(jnp.float32)
        cs = jnp.dot(tri, oh, preferred_element_type=jnp.float32)
        cnt = cs[127:128, :]
        pre = jnp.dot(cnt, tric, preferred_element_type=jnp.float32)
        tot = jnp.sum(cnt).astype(jnp.int32)
        start.append(run)
        count.append(tot)
        run = run + tot
        inv_acc = inv_acc + oh * (jnp.float32(1.0) * start[e] + pre + cs - 1.0)
    inv_ref[...] = inv_acc.astype(jnp.int32)

    def schedule(p):
        eid = jnp.zeros((1, 16), jnp.int32)
        cum = jnp.int32(0)
        nvs = []
        for e in range(E):
            s_e, c_e = start[e], count[e]
            ft = lax.div(s_e, jnp.int32(TILE))
            lt = lax.div(s_e + c_e - 1, jnp.int32(TILE))
            nv = jnp.where(c_e > 0, lt - ft + 1, 0)
            nvs.append((ft, nv))
            cum = cum + nv
            if e < E - 1:
                eid = eid + jnp.where(cum <= p, 1, 0)
        tile = jnp.zeros((1, 16), jnp.int32)
        lo = jnp.zeros((1, 16), jnp.int32)
        hi = jnp.zeros((1, 16), jnp.int32)
        vcum = jnp.int32(0)
        for e in range(E):
            ft, nv = nvs[e]
            m = eid == e
            t_e = ft + (p - vcum)
            tile = jnp.where(m, t_e, tile)
            lo = jnp.where(m, jnp.maximum(start[e], t_e * TILE), lo)
            hi = jnp.where(m, jnp.minimum(start[e] + count[e], (t_e + 1) * TILE), hi)
            vcum = vcum + nv
        valid = p < cum
        tile = jnp.where(valid, tile, NT - 1)
        lo = jnp.where(valid, lo, 0)
        hi = jnp.where(valid, hi, 0)
        return eid, tile, lo, hi

    p = lax.broadcasted_iota(jnp.int32, (1, 16), 1)
    eid, tile, lo, hi = schedule(p)
    _, tile_prev, _, _ = schedule(p - 1)
    fresh = ((p == 0) | (tile != tile_prev)).astype(jnp.int32)
    meta_ref[...] = jnp.concatenate([tile, eid, lo, hi, fresh], axis=0)


def _tc_route_metadata(g):
    inv2d, meta = pl.pallas_call(
        _meta_body,
        in_specs=[pl.BlockSpec((128, 32), lambda: (0, 0))],
        out_specs=[
            pl.BlockSpec((128, 32), lambda: (0, 0)),
            pl.BlockSpec((5, 16), lambda: (0, 0)),
        ],
        out_shape=[
            jax.ShapeDtypeStruct((128, 32), jnp.int32),
            jax.ShapeDtypeStruct((5, 16), jnp.int32),
        ],
    )(g.reshape(128, 32))
    return inv2d.reshape(T), meta


def kernel(x, groups, W, b):
    g = groups[:, 0]
    inv_idx, meta = _tc_route_metadata(g)
    x_sorted = _sc_permute_rows(x, inv_idx, "scatter")
    out_sorted = _grouped_matmul(x_sorted, W, b, meta)
    return _sc_permute_rows(out_sorted, inv_idx, "gather")
